# baseline (speedup 1.0000x reference)
import jax
import jax.numpy as jnp
from jax import lax
from jax.experimental import pallas as pl
from jax.experimental.pallas import tpu as pltpu
from jax.experimental.pallas import tpu_sc as plsc

_VOCAB = 50257
_H = 2048
_B = 4
_S = 2048

_NC = 2
_NS = 16
_NW = _NC * _NS
_ROWS = _B * _S
_RPW = _ROWS // _NW
_CH = 8
_NCHUNK = _RPW // _CH
_NSLOT = 3
_LANES = 16


def _body(ids_hbm, pids_hbm, wtab_hbm, ptab_hbm, out_hbm,
          blk_v, widx_v, pidx_v,
          wbuf0, wbuf1, wbuf2, pbuf0, pbuf1, pbuf2,
          gw0, gw1, gw2, gp0, gp1, gp2, go0, go1, go2):
    wbufs = (wbuf0, wbuf1, wbuf2)
    pbufs = (pbuf0, pbuf1, pbuf2)
    gw_sems = (gw0, gw1, gw2)
    gp_sems = (gp0, gp1, gp2)
    go_sems = (go0, go1, go2)

    cid = lax.axis_index("c")
    sid = lax.axis_index("s")
    wid = sid * _NC + cid
    row0 = wid * _RPW
    s_base = row0 // _B

    a0 = (wid // 2) * 128
    off = (wid % 2) * (_RPW // _B)

    def interleave(src_hbm, dst_v):
        pltpu.sync_copy(src_hbm.at[:, pl.ds(a0, 128)], blk_v)
        for g in range(_RPW // _LANES):
            p = lax.iota(jnp.int32, _LANES) + (g * _LANES)
            rows = lax.bitwise_and(p, _B - 1)
            cols = off + lax.shift_right_logical(p, 2)
            dst_v[pl.ds(g * _LANES, _LANES)] = plsc.load_gather(
                blk_v, [rows, cols])

    interleave(ids_hbm, widx_v)
    interleave(pids_hbm, pidx_v)

    def issue_gathers(h):
        sl = h % _NSLOT
        dw = pltpu.async_copy(
            wtab_hbm.at[widx_v.at[pl.ds(h * _CH, _CH)]], wbufs[sl],
            gw_sems[sl])
        dp = pltpu.async_copy(
            ptab_hbm.at[pidx_v.at[pl.ds(h * _CH, _CH)]], pbufs[sl],
            gp_sems[sl])
        return dw, dp

    def do_add(sl):
        wb, pb = wbufs[sl], pbufs[sl]
        unroll = 8

        def outer(i, carry):
            r = lax.shift_right_logical(i, 4)
            base = lax.shift_left(lax.bitwise_and(i, 15), 7)
            for u in range(unroll):
                c = pl.multiple_of(base + u * _LANES, _LANES)
                plsc.addupdate(wb.at[r, pl.ds(c, _LANES)],
                               pb[r, pl.ds(c, _LANES)])
            return carry

        lax.fori_loop(0, _CH * (_H // (_LANES * unroll)), outer, None)

    pend = {}
    pend_out = {}
    for h in range(2):
        pend[h % _NSLOT] = issue_gathers(h)
    for g in range(_NCHUNK):
        sl = g % _NSLOT
        dw, dp = pend.pop(sl)
        dw.wait()
        dp.wait()
        do_add(sl)
        s = s_base + g * (_CH // _B)
        d0 = pltpu.async_copy(wbufs[sl].at[pl.ds(0, _B)], out_hbm.at[s],
                              go_sems[sl])
        d1 = pltpu.async_copy(wbufs[sl].at[pl.ds(_B, _B)], out_hbm.at[s + 1],
                              go_sems[sl])
        pend_out[sl] = (d0, d1)
        h = g + 2
        if h < _NCHUNK:
            hs = h % _NSLOT
            if hs in pend_out:
                for d in pend_out.pop(hs):
                    d.wait()
            pend[hs] = issue_gathers(h)
    for sl in sorted(pend_out):
        for d in pend_out.pop(sl):
            d.wait()


@jax.jit
def _embed(input_ids, position_ids, word_embeddings, position_embeddings):
    mesh = plsc.VectorSubcoreMesh(core_axis_name="c", subcore_axis_name="s")
    scratch = [
        pltpu.VMEM((_B, 128), jnp.int32),
        pltpu.VMEM((_RPW,), jnp.int32),
        pltpu.VMEM((_RPW,), jnp.int32),
    ]
    scratch += [pltpu.VMEM((_CH, _H), jnp.float32) for _ in range(2 * _NSLOT)]
    scratch += [pltpu.SemaphoreType.DMA for _ in range(3 * _NSLOT)]
    run = pl.kernel(
        _body,
        out_type=jax.ShapeDtypeStruct((_S, _B, _H), jnp.float32),
        mesh=mesh,
        scratch_types=scratch,
        compiler_params=pltpu.CompilerParams(use_tc_tiling_on_sc=True,
                                             needs_layout_passes=False),
    )
    return run(input_ids, position_ids, word_embeddings,
               position_embeddings)


def kernel(input_ids, position_ids, word_embeddings, position_embeddings):
    return _embed(input_ids.astype(jnp.int32), position_ids.astype(jnp.int32),
                  word_embeddings, position_embeddings)


# speedup vs baseline: 1.0153x; 1.0153x over previous
import jax
import jax.numpy as jnp
from jax import lax
from jax.experimental import pallas as pl
from jax.experimental.pallas import tpu as pltpu
from jax.experimental.pallas import tpu_sc as plsc

_VOCAB = 50257
_H = 2048
_B = 4
_S = 2048

_NC = 2
_NS = 16
_NW = _NC * _NS
_ROWS = _B * _S
_RPW = _ROWS // _NW
_CH = 8
_NCHUNK = _RPW // _CH
_NSLOT = 3
_LANES = 16


def _body(ids_hbm, pids_hbm, wtab_hbm, ptab_hbm, out_hbm,
          blk_v, widx_v, pidx_v,
          wbuf0, wbuf1, wbuf2, pbuf0, pbuf1, pbuf2,
          gw0, gw1, gw2, gp0, gp1, gp2, go0, go1, go2):
    wbufs = (wbuf0, wbuf1, wbuf2)
    pbufs = (pbuf0, pbuf1, pbuf2)
    gw_sems = (gw0, gw1, gw2)
    gp_sems = (gp0, gp1, gp2)
    go_sems = (go0, go1, go2)

    cid = lax.axis_index("c")
    sid = lax.axis_index("s")
    wid = sid * _NC + cid
    row0 = wid * _RPW
    s_base = row0 // _B

    a0 = (wid // 2) * 128
    off = (wid % 2) * (_RPW // _B)

    def interleave(src_hbm, dst_v):
        pltpu.sync_copy(src_hbm.at[:, pl.ds(a0, 128)], blk_v)
        for g in range(_RPW // _LANES):
            p = lax.iota(jnp.int32, _LANES) + (g * _LANES)
            rows = lax.bitwise_and(p, _B - 1)
            cols = off + lax.shift_right_logical(p, 2)
            dst_v[pl.ds(g * _LANES, _LANES)] = plsc.load_gather(
                blk_v, [rows, cols])

    interleave(ids_hbm, widx_v)
    interleave(pids_hbm, pidx_v)

    def issue_gathers(h):
        sl = h % _NSLOT
        dw = pltpu.async_copy(
            wtab_hbm.at[widx_v.at[pl.ds(h * _CH, _CH)]], wbufs[sl],
            gw_sems[sl])
        dp = pltpu.async_copy(
            ptab_hbm.at[pidx_v.at[pl.ds(h * _CH, _CH)]], pbufs[sl],
            gp_sems[sl])
        return dw, dp

    def do_add(sl):
        wb, pb = wbufs[sl], pbufs[sl]
        unroll = 8

        def outer(i, carry):
            r = lax.shift_right_logical(i, 4)
            base = lax.shift_left(lax.bitwise_and(i, 15), 7)
            for u in range(unroll):
                c = pl.multiple_of(base + u * _LANES, _LANES)
                plsc.addupdate(wb.at[r, pl.ds(c, _LANES)],
                               pb[r, pl.ds(c, _LANES)])
            return carry

        lax.fori_loop(0, _CH * (_H // (_LANES * unroll)), outer, None)

    pend = {}
    pend_out = {}
    for h in range(2):
        pend[h % _NSLOT] = issue_gathers(h)
    for g in range(_NCHUNK):
        sl = g % _NSLOT
        dw, dp = pend.pop(sl)
        dw.wait()
        dp.wait()
        h = g + 2
        if h < _NCHUNK:
            hs = h % _NSLOT
            if hs in pend_out:
                for d in pend_out.pop(hs):
                    d.wait()
            pend[hs] = issue_gathers(h)
        do_add(sl)
        s = s_base + g * (_CH // _B)
        d0 = pltpu.async_copy(wbufs[sl].at[pl.ds(0, _B)], out_hbm.at[s],
                              go_sems[sl])
        d1 = pltpu.async_copy(wbufs[sl].at[pl.ds(_B, _B)], out_hbm.at[s + 1],
                              go_sems[sl])
        pend_out[sl] = (d0, d1)
    for sl in sorted(pend_out):
        for d in pend_out.pop(sl):
            d.wait()


@jax.jit
def _embed(input_ids, position_ids, word_embeddings, position_embeddings):
    mesh = plsc.VectorSubcoreMesh(core_axis_name="c", subcore_axis_name="s")
    scratch = [
        pltpu.VMEM((_B, 128), jnp.int32),
        pltpu.VMEM((_RPW,), jnp.int32),
        pltpu.VMEM((_RPW,), jnp.int32),
    ]
    scratch += [pltpu.VMEM((_CH, _H), jnp.float32) for _ in range(2 * _NSLOT)]
    scratch += [pltpu.SemaphoreType.DMA for _ in range(3 * _NSLOT)]
    run = pl.kernel(
        _body,
        out_type=jax.ShapeDtypeStruct((_S, _B, _H), jnp.float32),
        mesh=mesh,
        scratch_types=scratch,
        compiler_params=pltpu.CompilerParams(use_tc_tiling_on_sc=True,
                                             needs_layout_passes=False),
    )
    return run(input_ids, position_ids, word_embeddings,
               position_embeddings)


def kernel(input_ids, position_ids, word_embeddings, position_embeddings):
    return _embed(input_ids.astype(jnp.int32), position_ids.astype(jnp.int32),
                  word_embeddings, position_embeddings)
